# baseline (speedup 1.0000x reference)
import jax
import jax.numpy as jnp
from jax import lax
from jax.experimental import pallas as pl
from jax.experimental.pallas import tpu as pltpu
from jax.experimental.pallas import tpu_sc as plsc

_VOCAB = 100000
_NSEG = 2
_LEN = 200
_D = 64
_DP = 128
_B = 4096
_N = _B * _LEN

_NC, _NS, _L = 2, 16, 16
_NW = _NC * _NS
_NPW = _N // _NW
_C = 128
_CHUNKS = _NPW // _C
_HALF = _CHUNKS // 2


def _sinusoidal_encoding():
    position = jnp.arange(0, _LEN, dtype=jnp.float32)[:, None]
    div_term = jnp.exp(
        jnp.arange(0, _D, 2, dtype=jnp.float32)
        * -(jnp.log(jnp.asarray(10000.0)) / _D)
    )
    enc = jnp.zeros((_LEN, _D), dtype=jnp.float32)
    enc = enc.at[:, 0::2].set(jnp.sin(position * div_term))
    enc = enc.at[:, 1::2].set(jnp.cos(position * div_term))
    return enc


def _body(seq_hbm, seg_hbm, tok_hbm, add_hbm, out_hbm,
          add_tab,
          seq_idx0, aidx0, tok0, wb0,
          seq_idx1, aidx1, tok1, wb1,
          st0, sw0, st1, sw1):
    sets = (
        (seq_idx0, aidx0, tok0, wb0, st0, sw0),
        (seq_idx1, aidx1, tok1, wb1, st1, sw1),
    )
    wid = lax.axis_index("s") * _NC + lax.axis_index("c")
    wbase = wid * _NPW
    obase = wid * (_NPW // 2)
    lanes = lax.iota(jnp.int32, _L)

    pltpu.sync_copy(add_hbm, add_tab)

    def stage(cc, bufs):
        seq_idx, aidx, tok_buf, _, sem_tok, _ = bufs
        base = wbase + cc * _C
        pltpu.sync_copy(seq_hbm.at[pl.ds(base, _C)], seq_idx)
        pltpu.sync_copy(seg_hbm.at[pl.ds(base, _C)], aidx)
        for j in range(_C // _L):
            sl = pl.ds(j * _L, _L)
            pos = lax.rem(base + j * _L + lanes, _LEN)
            aidx[sl] = aidx[sl] * _LEN + pos
        pltpu.async_copy(tok_hbm.at[seq_idx], tok_buf, sem_tok)

    stage(0, sets[0])
    stage(1, sets[1])

    def pair(i, carry):
        for s in range(2):
            bufs = sets[s]
            seq_idx, aidx, tok_buf, wb_buf, sem_tok, sem_wb = bufs
            cc = 2 * i + s
            ob = obase + cc * (_C // 2)
            pltpu.make_async_copy(tok_hbm.at[seq_idx], tok_buf, sem_tok).wait()

            @pl.when(i >= 1)
            def _wait_wb():
                pltpu.make_async_copy(
                    wb_buf, out_hbm.at[pl.ds(ob, _C // 2)], sem_wb).wait()

            @plsc.parallel_loop(0, _C // _L, 1)
            def _combine(g):
                a_vec = aidx[pl.ds(g * _L, _L)]
                for h in range(_L):
                    r = g * _L + h
                    a = a_vec[h]
                    for j in range(_D // _L):
                        t = tok_buf[r, pl.ds(j * _L, _L)]
                        av = add_tab[a, pl.ds(j * _L, _L)]
                        wb_buf[g * (_L // 2) + h // 2,
                               pl.ds((h % 2) * _D + j * _L, _L)] = av + t + t

            pltpu.async_copy(wb_buf, out_hbm.at[pl.ds(ob, _C // 2)], sem_wb)

            @pl.when(i < _HALF - 1)
            def _prefetch():
                stage(cc + 2, bufs)

        return carry

    lax.fori_loop(0, _HALF, pair, 0)
    for s in range(2):
        wb_buf, sem_wb = sets[s][3], sets[s][5]
        pltpu.make_async_copy(
            wb_buf, out_hbm.at[pl.ds(obase, _C // 2)], sem_wb).wait()


_sc_call = pl.kernel(
    _body,
    out_type=jax.ShapeDtypeStruct((_N // 2, _DP), jnp.float32),
    mesh=plsc.VectorSubcoreMesh(core_axis_name="c", subcore_axis_name="s"),
    scratch_types=[
        pltpu.VMEM((_NSEG * _LEN, _DP), jnp.float32),
        pltpu.VMEM((_C,), jnp.int32),
        pltpu.VMEM((_C,), jnp.int32),
        pltpu.VMEM((_C, _DP), jnp.float32),
        pltpu.VMEM((_C // 2, _DP), jnp.float32),
        pltpu.VMEM((_C,), jnp.int32),
        pltpu.VMEM((_C,), jnp.int32),
        pltpu.VMEM((_C, _DP), jnp.float32),
        pltpu.VMEM((_C // 2, _DP), jnp.float32),
        pltpu.SemaphoreType.DMA,
        pltpu.SemaphoreType.DMA,
        pltpu.SemaphoreType.DMA,
        pltpu.SemaphoreType.DMA,
    ],
    compiler_params=pltpu.CompilerParams(use_tc_tiling_on_sc=True),
)


@jax.jit
def kernel(seq, seg, tok_table, seg_table):
    enc = _sinusoidal_encoding()
    addend = (seg_table[:, None, :] + enc[None, :, :]).reshape(
        _NSEG * _LEN, _D)
    addend_p = jnp.pad(addend, ((0, 0), (0, _DP - _D)))
    tok_p = jnp.pad(tok_table, ((0, 0), (0, _DP - _D)))
    seq_f = seq.reshape(_N).astype(jnp.int32)
    seg_f = seg.reshape(_N).astype(jnp.int32)
    out = _sc_call(seq_f, seg_f, tok_p, addend_p)
    return out.reshape(_B, _LEN, _D)


# speedup vs baseline: 1.0222x; 1.0222x over previous
import jax
import jax.numpy as jnp
from jax import lax
from jax.experimental import pallas as pl
from jax.experimental.pallas import tpu as pltpu
from jax.experimental.pallas import tpu_sc as plsc

_VOCAB = 100000
_NSEG = 2
_LEN = 200
_D = 64
_B = 4096
_N = _B * _LEN

_NC, _NS, _L = 2, 16, 16
_NW = _NC * _NS
_NPW = _N // _NW
_C = 128
_CHUNKS = _NPW // _C
_HALF = _CHUNKS // 2


def _sinusoidal_encoding():
    position = jnp.arange(0, _LEN, dtype=jnp.float32)[:, None]
    div_term = jnp.exp(
        jnp.arange(0, _D, 2, dtype=jnp.float32)
        * -(jnp.log(jnp.asarray(10000.0)) / _D)
    )
    enc = jnp.zeros((_LEN, _D), dtype=jnp.float32)
    enc = enc.at[:, 0::2].set(jnp.sin(position * div_term))
    enc = enc.at[:, 1::2].set(jnp.cos(position * div_term))
    return enc


def _body(seq_hbm, seg_hbm, tok_hbm, add_hbm, out_hbm,
          add_tab,
          seq0, aidx0, tok0, wb0,
          seq1, aidx1, tok1, wb1,
          st0, sw0, st1, sw1):
    sets = (
        (seq0, aidx0, tok0, wb0, st0, sw0),
        (seq1, aidx1, tok1, wb1, st1, sw1),
    )
    wid = lax.axis_index("s") * _NC + lax.axis_index("c")
    wbase = wid * _NPW
    lanes = lax.iota(jnp.int32, _L)

    pltpu.sync_copy(add_hbm, add_tab)

    def stage(cc, bufs):
        seq_idx, aidx, tok_buf, _, sem_tok, _ = bufs
        base = wbase + cc * _C
        pltpu.sync_copy(seq_hbm.at[pl.ds(base, _C)], seq_idx)
        pltpu.sync_copy(seg_hbm.at[pl.ds(base, _C)], aidx)
        for j in range(_C // _L):
            sl = pl.ds(j * _L, _L)
            pos = lax.rem(base + j * _L + lanes, _LEN)
            aidx[sl] = aidx[sl] * _LEN + pos
        pltpu.async_copy(tok_hbm.at[seq_idx], tok_buf, sem_tok)

    stage(0, sets[0])
    stage(1, sets[1])

    def pair(i, carry):
        for s in range(2):
            bufs = sets[s]
            seq_idx, aidx, tok_buf, wb_buf, sem_tok, sem_wb = bufs
            cc = 2 * i + s
            base = wbase + cc * _C
            pltpu.make_async_copy(tok_hbm.at[seq_idx], tok_buf, sem_tok).wait()

            @pl.when(i >= 1)
            def _wait_wb():
                pltpu.make_async_copy(
                    wb_buf, out_hbm.at[pl.ds(base, _C)], sem_wb).wait()

            @plsc.parallel_loop(0, _C // _L, 1)
            def _combine(g):
                a_vec = aidx[pl.ds(g * _L, _L)]
                for h in range(_L):
                    r = g * _L + h
                    a = a_vec[h]
                    for j in range(_D // _L):
                        sl = pl.ds(j * _L, _L)
                        t = tok_buf[r, sl]
                        av = add_tab[a, sl]
                        wb_buf[r, sl] = av + t + t

            pltpu.async_copy(wb_buf, out_hbm.at[pl.ds(base, _C)], sem_wb)

            @pl.when(i < _HALF - 1)
            def _prefetch():
                stage(cc + 2, bufs)

        return carry

    lax.fori_loop(0, _HALF, pair, 0)
    for s in range(2):
        wb_buf, sem_wb = sets[s][3], sets[s][5]
        pltpu.make_async_copy(
            wb_buf, out_hbm.at[pl.ds(wbase, _C)], sem_wb).wait()


_sc_call = pl.kernel(
    _body,
    out_type=jax.ShapeDtypeStruct((_N, _D), jnp.float32),
    mesh=plsc.VectorSubcoreMesh(core_axis_name="c", subcore_axis_name="s"),
    scratch_types=[
        pltpu.VMEM((_NSEG * _LEN, _D), jnp.float32),
        pltpu.VMEM((_C,), jnp.int32),
        pltpu.VMEM((_C,), jnp.int32),
        pltpu.VMEM((_C, _D), jnp.float32),
        pltpu.VMEM((_C, _D), jnp.float32),
        pltpu.VMEM((_C,), jnp.int32),
        pltpu.VMEM((_C,), jnp.int32),
        pltpu.VMEM((_C, _D), jnp.float32),
        pltpu.VMEM((_C, _D), jnp.float32),
        pltpu.SemaphoreType.DMA,
        pltpu.SemaphoreType.DMA,
        pltpu.SemaphoreType.DMA,
        pltpu.SemaphoreType.DMA,
    ],
    compiler_params=pltpu.CompilerParams(use_tc_tiling_on_sc=False),
)


@jax.jit
def kernel(seq, seg, tok_table, seg_table):
    enc = _sinusoidal_encoding()
    addend = (seg_table[:, None, :] + enc[None, :, :]).reshape(
        _NSEG * _LEN, _D)
    seq_f = seq.reshape(_N).astype(jnp.int32)
    seg_f = seg.reshape(_N).astype(jnp.int32)
    out = _sc_call(seq_f, seg_f, tok_table, addend)
    return out.reshape(_B, _LEN, _D)


# speedup vs baseline: 1.3480x; 1.3187x over previous
import jax
import jax.numpy as jnp
from jax import lax
from jax.experimental import pallas as pl
from jax.experimental.pallas import tpu as pltpu
from jax.experimental.pallas import tpu_sc as plsc

_VOCAB = 100000
_NSEG = 2
_LEN = 200
_D = 64
_B = 4096
_N = _B * _LEN

_NC, _NS, _L = 2, 16, 16
_NW = _NC * _NS
_NPW = _N // _NW
_C = 128
_CHUNKS = _NPW // _C
_HALF = _CHUNKS // 2


def _sinusoidal_encoding():
    position = jnp.arange(0, _LEN, dtype=jnp.float32)[:, None]
    div_term = jnp.exp(
        jnp.arange(0, _D, 2, dtype=jnp.float32)
        * -(jnp.log(jnp.asarray(10000.0)) / _D)
    )
    enc = jnp.zeros((_LEN, _D), dtype=jnp.float32)
    enc = enc.at[:, 0::2].set(jnp.sin(position * div_term))
    enc = enc.at[:, 1::2].set(jnp.cos(position * div_term))
    return enc


def _body(seq_hbm, seg_hbm, tok_hbm, add_hbm, out_hbm,
          add_tab, seq_all, aidx_all,
          tok0, wb0, tok1, wb1,
          st0, sw0, st1, sw1):
    sets = (
        (tok0, wb0, st0, sw0),
        (tok1, wb1, st1, sw1),
    )
    wid = lax.axis_index("s") * _NC + lax.axis_index("c")
    wbase = wid * _NPW
    lanes = lax.iota(jnp.int32, _L)

    pltpu.sync_copy(add_hbm, add_tab)
    pltpu.sync_copy(seq_hbm.at[pl.ds(wbase, _NPW)], seq_all)
    pltpu.sync_copy(seg_hbm.at[pl.ds(wbase, _NPW)], aidx_all)

    @plsc.parallel_loop(0, _NPW // _L, 1)
    def _mk_aidx(g):
        sl = pl.ds(g * _L, _L)
        pos = lax.rem(wbase + g * _L + lanes, _LEN)
        aidx_all[sl] = aidx_all[sl] * _LEN + pos

    def issue_tok(cc, bufs):
        tok_buf, _, sem_tok, _ = bufs
        pltpu.async_copy(
            tok_hbm.at[seq_all.at[pl.ds(cc * _C, _C)]], tok_buf, sem_tok)

    issue_tok(0, sets[0])
    issue_tok(1, sets[1])

    def pair(i, carry):
        for s in range(2):
            bufs = sets[s]
            tok_buf, wb_buf, sem_tok, sem_wb = bufs
            cc = 2 * i + s
            base = wbase + cc * _C

            @pl.when(i >= 1)
            def _wait_wb():
                pltpu.make_async_copy(
                    wb_buf, out_hbm.at[pl.ds(base, _C)], sem_wb).wait()

            pltpu.make_async_copy(
                tok_hbm.at[seq_all.at[pl.ds(cc * _C, _C)]],
                tok_buf, sem_tok).wait()

            @plsc.parallel_loop(0, _C // _L, 1)
            def _combine(g):
                a_vec = aidx_all[pl.ds(cc * _C + g * _L, _L)]
                for h in range(_L):
                    r = g * _L + h
                    a = a_vec[h]
                    for j in range(_D // _L):
                        sl = pl.ds(j * _L, _L)
                        t = tok_buf[r, sl]
                        wb_buf[r, sl] = add_tab[a, sl] + t + t

            pltpu.async_copy(wb_buf, out_hbm.at[pl.ds(base, _C)], sem_wb)

            @pl.when(i < _HALF - 1)
            def _prefetch():
                issue_tok(cc + 2, bufs)

        return carry

    lax.fori_loop(0, _HALF, pair, 0)
    for s in range(2):
        wb_buf, sem_wb = sets[s][1], sets[s][3]
        pltpu.make_async_copy(
            wb_buf, out_hbm.at[pl.ds(wbase, _C)], sem_wb).wait()


_sc_call = pl.kernel(
    _body,
    out_type=jax.ShapeDtypeStruct((_N, _D), jnp.float32),
    mesh=plsc.VectorSubcoreMesh(core_axis_name="c", subcore_axis_name="s"),
    scratch_types=[
        pltpu.VMEM((_NSEG * _LEN, _D), jnp.float32),
        pltpu.VMEM((_NPW,), jnp.int32),
        pltpu.VMEM((_NPW,), jnp.int32),
        pltpu.VMEM((_C, _D), jnp.float32),
        pltpu.VMEM((_C, _D), jnp.float32),
        pltpu.VMEM((_C, _D), jnp.float32),
        pltpu.VMEM((_C, _D), jnp.float32),
        pltpu.SemaphoreType.DMA,
        pltpu.SemaphoreType.DMA,
        pltpu.SemaphoreType.DMA,
        pltpu.SemaphoreType.DMA,
    ],
    compiler_params=pltpu.CompilerParams(use_tc_tiling_on_sc=False),
)


@jax.jit
def kernel(seq, seg, tok_table, seg_table):
    enc = _sinusoidal_encoding()
    addend = (seg_table[:, None, :] + enc[None, :, :]).reshape(
        _NSEG * _LEN, _D)
    seq_f = seq.reshape(_N).astype(jnp.int32)
    seg_f = seg.reshape(_N).astype(jnp.int32)
    out = _sc_call(seq_f, seg_f, tok_table, addend)
    return out.reshape(_B, _LEN, _D)


# speedup vs baseline: 1.4177x; 1.0517x over previous
import jax
import jax.numpy as jnp
from jax import lax
from jax.experimental import pallas as pl
from jax.experimental.pallas import tpu as pltpu
from jax.experimental.pallas import tpu_sc as plsc

_VOCAB = 100000
_NSEG = 2
_LEN = 200
_D = 64
_B = 4096

_NC, _NS, _L = 2, 16, 16
_NW = _NC * _NS
_BPW = _B // _NW
_HALF = _BPW // 2
_G0 = 128
_G1 = _LEN - _G0
_FULL = (_LEN // _L) * _L
_LP = 208


def _sinusoidal_encoding():
    position = jnp.arange(0, _LEN, dtype=jnp.float32)[:, None]
    div_term = jnp.exp(
        jnp.arange(0, _D, 2, dtype=jnp.float32)
        * -(jnp.log(jnp.asarray(10000.0)) / _D)
    )
    enc = jnp.zeros((_LEN, _D), dtype=jnp.float32)
    enc = enc.at[:, 0::2].set(jnp.sin(position * div_term))
    enc = enc.at[:, 1::2].set(jnp.cos(position * div_term))
    return enc


def _body(seq_hbm, seg_hbm, tok_hbm, add_hbm, out_hbm,
          add_tab, seq_all, aidx_all,
          tok0, wb0, tok1, wb1,
          st0, sw0, st1, sw1):
    sets = (
        (tok0, wb0, st0, sw0),
        (tok1, wb1, st1, sw1),
    )
    wid = lax.axis_index("s") * _NC + lax.axis_index("c")
    bbase = wid * _BPW
    lanes = lax.iota(jnp.int32, _L)

    pltpu.sync_copy(add_hbm, add_tab)
    pltpu.sync_copy(seq_hbm.at[pl.ds(bbase, _BPW)], seq_all)
    pltpu.sync_copy(
        seg_hbm.at[pl.ds(bbase, _BPW)],
        aidx_all.at[pl.ds(0, _BPW), pl.ds(0, _LEN)])

    @plsc.parallel_loop(0, _BPW, 1)
    def _mk_aidx(r):
        for g in range(_LP // _L):
            sl = pl.ds(g * _L, _L)
            aidx_all[r, sl] = aidx_all[r, sl] * _LEN + (g * _L + lanes)

    def issue_tok(cc, bufs):
        tok_buf, _, sem_tok, _ = bufs
        pltpu.async_copy(
            tok_hbm.at[seq_all.at[cc, pl.ds(0, _G0)]],
            tok_buf.at[pl.ds(0, _G0)], sem_tok)
        pltpu.async_copy(
            tok_hbm.at[seq_all.at[cc, pl.ds(_G0, _G1)]],
            tok_buf.at[pl.ds(_G0, _G1)], sem_tok)

    issue_tok(0, sets[0])
    issue_tok(1, sets[1])

    def pair(i, carry):
        for s in range(2):
            bufs = sets[s]
            tok_buf, wb_buf, sem_tok, sem_wb = bufs
            cc = 2 * i + s
            b = bbase + cc

            @pl.when(i >= 1)
            def _wait_wb():
                pltpu.make_async_copy(wb_buf, out_hbm.at[b], sem_wb).wait()

            pltpu.make_async_copy(
                tok_hbm.at[seq_all.at[cc, pl.ds(0, _G0)]],
                tok_buf.at[pl.ds(0, _G0)], sem_tok).wait()
            pltpu.make_async_copy(
                tok_hbm.at[seq_all.at[cc, pl.ds(_G0, _G1)]],
                tok_buf.at[pl.ds(_G0, _G1)], sem_tok).wait()

            @plsc.parallel_loop(0, _LEN // _L, 1)
            def _combine(g):
                a_vec = aidx_all[cc, pl.ds(g * _L, _L)]
                for h in range(_L):
                    r = g * _L + h
                    a = a_vec[h]
                    for j in range(_D // _L):
                        sl = pl.ds(j * _L, _L)
                        t = tok_buf[r, sl]
                        wb_buf[r, sl] = add_tab[a, sl] + t + t

            a_vec = aidx_all[cc, pl.ds(_LEN - _L, _L)]
            for h in range(_L - (_LEN - _FULL), _L):
                r = _LEN - _L + h
                a = a_vec[h]
                for j in range(_D // _L):
                    sl = pl.ds(j * _L, _L)
                    t = tok_buf[r, sl]
                    wb_buf[r, sl] = add_tab[a, sl] + t + t

            pltpu.async_copy(wb_buf, out_hbm.at[b], sem_wb)

            @pl.when(i < _HALF - 1)
            def _prefetch():
                issue_tok(cc + 2, bufs)

        return carry

    lax.fori_loop(0, _HALF, pair, 0)
    for s in range(2):
        wb_buf, sem_wb = sets[s][1], sets[s][3]
        pltpu.make_async_copy(wb_buf, out_hbm.at[bbase], sem_wb).wait()


_sc_call = pl.kernel(
    _body,
    out_type=jax.ShapeDtypeStruct((_B, _LEN, _D), jnp.float32),
    mesh=plsc.VectorSubcoreMesh(core_axis_name="c", subcore_axis_name="s"),
    scratch_types=[
        pltpu.VMEM((_NSEG * _LEN, _D), jnp.float32),
        pltpu.VMEM((_BPW, _LEN), jnp.int32),
        pltpu.VMEM((_BPW, _LP), jnp.int32),
        pltpu.VMEM((_LEN, _D), jnp.float32),
        pltpu.VMEM((_LEN, _D), jnp.float32),
        pltpu.VMEM((_LEN, _D), jnp.float32),
        pltpu.VMEM((_LEN, _D), jnp.float32),
        pltpu.SemaphoreType.DMA,
        pltpu.SemaphoreType.DMA,
        pltpu.SemaphoreType.DMA,
        pltpu.SemaphoreType.DMA,
    ],
    compiler_params=pltpu.CompilerParams(use_tc_tiling_on_sc=False),
)


@jax.jit
def kernel(seq, seg, tok_table, seg_table):
    enc = _sinusoidal_encoding()
    addend = (seg_table[:, None, :] + enc[None, :, :]).reshape(
        _NSEG * _LEN, _D)
    return _sc_call(seq.astype(jnp.int32), seg.astype(jnp.int32),
                    tok_table, addend)


# speedup vs baseline: 1.4776x; 1.0423x over previous
import jax
import jax.numpy as jnp
from jax import lax
from jax.experimental import pallas as pl
from jax.experimental.pallas import tpu as pltpu
from jax.experimental.pallas import tpu_sc as plsc

_VOCAB = 100000
_NSEG = 2
_LEN = 200
_D = 64
_B = 4096
_N = _B * _LEN
_TOKP = _VOCAB // 2

_NC, _NS, _L = 2, 16, 16
_NW = _NC * _NS
_BPW = _B // _NW
_NPW = _BPW * _LEN
_HALF = _BPW // 2
_G0 = 128
_G1 = _LEN - _G0
_FULL = (_LEN // _L) * _L


def _sinusoidal_encoding():
    position = jnp.arange(0, _LEN, dtype=jnp.float32)[:, None]
    div_term = jnp.exp(
        jnp.arange(0, _D, 2, dtype=jnp.float32)
        * -(jnp.log(jnp.asarray(10000.0)) / _D)
    )
    enc = jnp.zeros((_LEN, _D), dtype=jnp.float32)
    enc = enc.at[:, 0::2].set(jnp.sin(position * div_term))
    enc = enc.at[:, 1::2].set(jnp.cos(position * div_term))
    return enc


def _body(seqp_hbm, code_hbm, tok_hbm, add_hbm, out_hbm,
          add_tab,
          tok0, wb0, seqp0, code0,
          tok1, wb1, seqp1, code1,
          si0, st0, sw0, si1, st1, sw1):
    sets = (
        (tok0, wb0, seqp0, code0, si0, st0, sw0),
        (tok1, wb1, seqp1, code1, si1, st1, sw1),
    )
    wid = lax.axis_index("s") * _NC + lax.axis_index("c")
    bbase = wid * _BPW
    nbase = wid * _NPW

    pltpu.sync_copy(add_hbm, add_tab)

    def issue_idx(cc, bufs):
        _, _, seqp_buf, code_buf, sem_idx = bufs[:5]
        pltpu.async_copy(
            seqp_hbm.at[pl.ds(nbase + cc * _LEN, _LEN)], seqp_buf, sem_idx)
        pltpu.async_copy(
            code_hbm.at[pl.ds(nbase + cc * _LEN, _LEN)], code_buf, sem_idx)

    def issue_gather(bufs):
        tok_buf, _, seqp_buf, code_buf, sem_idx, sem_tok, _ = bufs
        pltpu.make_async_copy(
            seqp_hbm.at[pl.ds(nbase, _LEN)], seqp_buf, sem_idx).wait()
        pltpu.make_async_copy(
            code_hbm.at[pl.ds(nbase, _LEN)], code_buf, sem_idx).wait()
        pltpu.async_copy(
            tok_hbm.at[seqp_buf.at[pl.ds(0, _G0)]],
            tok_buf.at[pl.ds(0, _G0)], sem_tok)
        pltpu.async_copy(
            tok_hbm.at[seqp_buf.at[pl.ds(_G0, _G1)]],
            tok_buf.at[pl.ds(_G0, _G1)], sem_tok)

    def combine_rows(bufs, c_vec, rbase, hs):
        tok_buf, wb_buf = bufs[0], bufs[1]
        for h in range(hs, _L):
            r = rbase + h
            c = c_vec[h]
            o = (c >> 9) * _D
            a = (c & 511) * _D
            for j in range(_D // _L):
                sl = pl.ds(j * _L, _L)
                t = tok_buf[r, pl.ds(o + j * _L, _L)]
                wb_buf[r, sl] = add_tab[pl.ds(a + j * _L, _L)] + t + t

    def process(i, cc, bufs):
        tok_buf, wb_buf, seqp_buf, code_buf, _, sem_tok, sem_wb = bufs
        pltpu.make_async_copy(
            tok_hbm.at[seqp_buf.at[pl.ds(0, _G0)]],
            tok_buf.at[pl.ds(0, _G0)], sem_tok).wait()
        pltpu.make_async_copy(
            tok_hbm.at[seqp_buf.at[pl.ds(_G0, _G1)]],
            tok_buf.at[pl.ds(_G0, _G1)], sem_tok).wait()

        @pl.when(i >= 1)
        def _wait_wb():
            pltpu.make_async_copy(
                wb_buf, out_hbm.at[bbase + cc], sem_wb).wait()

        @plsc.parallel_loop(0, _FULL // _L, 1)
        def _combine(g):
            c_vec = code_buf[pl.ds(g * _L, _L)]
            combine_rows(bufs, c_vec, g * _L, 0)

        c_vec = code_buf[pl.ds(_LEN - _L, _L)]
        combine_rows(bufs, c_vec, _LEN - _L, _L - (_LEN - _FULL))

        pltpu.async_copy(wb_buf, out_hbm.at[bbase + cc], sem_wb)

    issue_idx(0, sets[0])
    issue_idx(1, sets[1])
    issue_gather(sets[0])

    def pair(i, carry):
        for s in range(2):
            bufs = sets[s]
            cc = 2 * i + s

            @pl.when(cc + 1 < _BPW)
            def _gather_next():
                issue_gather(sets[1 - s])

            process(i, cc, bufs)

            @pl.when(cc + 2 < _BPW)
            def _idx_next():
                issue_idx(cc + 2, bufs)

        return carry

    lax.fori_loop(0, _HALF, pair, 0)
    for s in range(2):
        wb_buf, sem_wb = sets[s][1], sets[s][6]
        pltpu.make_async_copy(wb_buf, out_hbm.at[bbase], sem_wb).wait()


_sc_call = pl.kernel(
    _body,
    out_type=jax.ShapeDtypeStruct((_B, _LEN, _D), jnp.float32),
    mesh=plsc.VectorSubcoreMesh(core_axis_name="c", subcore_axis_name="s"),
    scratch_types=[
        pltpu.VMEM((_NSEG * _LEN * _D,), jnp.float32),
        pltpu.VMEM((_LEN, 2 * _D), jnp.float32),
        pltpu.VMEM((_LEN, _D), jnp.float32),
        pltpu.VMEM((_LEN,), jnp.int32),
        pltpu.VMEM((_LEN,), jnp.int32),
        pltpu.VMEM((_LEN, 2 * _D), jnp.float32),
        pltpu.VMEM((_LEN, _D), jnp.float32),
        pltpu.VMEM((_LEN,), jnp.int32),
        pltpu.VMEM((_LEN,), jnp.int32),
        pltpu.SemaphoreType.DMA,
        pltpu.SemaphoreType.DMA,
        pltpu.SemaphoreType.DMA,
        pltpu.SemaphoreType.DMA,
        pltpu.SemaphoreType.DMA,
        pltpu.SemaphoreType.DMA,
    ],
    compiler_params=pltpu.CompilerParams(use_tc_tiling_on_sc=True),
)


@jax.jit
def kernel(seq, seg, tok_table, seg_table):
    enc = _sinusoidal_encoding()
    addend = (seg_table[:, None, :] + enc[None, :, :]).reshape(
        _NSEG * _LEN * _D)
    seq_i = seq.astype(jnp.int32)
    seqp = (seq_i >> 1).reshape(_N)
    code = (seg.astype(jnp.int32) * _LEN
            + jnp.arange(_LEN, dtype=jnp.int32)[None, :]
            + (seq_i & 1) * 512).reshape(_N)
    tok2 = tok_table.reshape(_TOKP, 2 * _D)
    return _sc_call(seqp, code, tok2, addend)


# speedup vs baseline: 1.5491x; 1.0484x over previous
import jax
import jax.numpy as jnp
from jax import lax
from jax.experimental import pallas as pl
from jax.experimental.pallas import tpu as pltpu
from jax.experimental.pallas import tpu_sc as plsc

_VOCAB = 100000
_NSEG = 2
_LEN = 200
_D = 64
_B = 4096
_N = _B * _LEN
_TOKP = _VOCAB // 2

_NC, _NS, _L = 2, 16, 16
_NW = _NC * _NS
_BPW = _B // _NW
_NPW = _BPW * _LEN
_HALF = _BPW // 2
_G0 = 128
_G1 = _LEN - _G0
_FULL = (_LEN // _L) * _L


def _sinusoidal_encoding():
    position = jnp.arange(0, _LEN, dtype=jnp.float32)[:, None]
    div_term = jnp.exp(
        jnp.arange(0, _D, 2, dtype=jnp.float32)
        * -(jnp.log(jnp.asarray(10000.0)) / _D)
    )
    enc = jnp.zeros((_LEN, _D), dtype=jnp.float32)
    enc = enc.at[:, 0::2].set(jnp.sin(position * div_term))
    enc = enc.at[:, 1::2].set(jnp.cos(position * div_term))
    return enc


def _body(seqp_hbm, code_hbm, tok_hbm, add_hbm, out_hbm,
          add_tab,
          tok0, wb0, seqp0, code0,
          tok1, wb1, seqp1, code1,
          si0, st0, sw0, si1, st1, sw1):
    sets = (
        (tok0, wb0, seqp0, code0, si0, st0, sw0),
        (tok1, wb1, seqp1, code1, si1, st1, sw1),
    )
    wid = lax.axis_index("s") * _NC + lax.axis_index("c")
    bbase = wid * _BPW
    nbase = wid * _NPW

    pltpu.sync_copy(add_hbm, add_tab)

    def issue_idx(cc, bufs):
        _, _, seqp_buf, code_buf, sem_idx = bufs[:5]
        pltpu.async_copy(
            seqp_hbm.at[pl.ds(nbase + cc * _LEN, _LEN)], seqp_buf, sem_idx)
        pltpu.async_copy(
            code_hbm.at[pl.ds(nbase + cc * _LEN, _LEN)], code_buf, sem_idx)

    def issue_gather(bufs):
        tok_buf, _, seqp_buf, code_buf, sem_idx, sem_tok, _ = bufs
        pltpu.make_async_copy(
            seqp_hbm.at[pl.ds(nbase, _LEN)], seqp_buf, sem_idx).wait()
        pltpu.make_async_copy(
            code_hbm.at[pl.ds(nbase, _LEN)], code_buf, sem_idx).wait()
        pltpu.async_copy(
            tok_hbm.at[seqp_buf.at[pl.ds(0, _G0)]],
            tok_buf.at[pl.ds(0, _G0)], sem_tok)
        pltpu.async_copy(
            tok_hbm.at[seqp_buf.at[pl.ds(_G0, _G1)]],
            tok_buf.at[pl.ds(_G0, _G1)], sem_tok)

    def combine_rows(bufs, c_vec, rbase, hs):
        tok_buf, wb_buf = bufs[0], bufs[1]
        for h in range(hs, _L):
            r = rbase + h
            c = c_vec[h]
            o = (c >> 9) * _D
            a = (c & 511) * _D
            for j in range(_D // _L):
                sl = pl.ds(j * _L, _L)
                t = tok_buf[r, pl.ds(o + j * _L, _L)]
                wb_buf[r, sl] = add_tab[pl.ds(a + j * _L, _L)] + t

    def process(i, cc, bufs):
        tok_buf, wb_buf, seqp_buf, code_buf, _, sem_tok, sem_wb = bufs
        pltpu.make_async_copy(
            tok_hbm.at[seqp_buf.at[pl.ds(0, _G0)]],
            tok_buf.at[pl.ds(0, _G0)], sem_tok).wait()
        pltpu.make_async_copy(
            tok_hbm.at[seqp_buf.at[pl.ds(_G0, _G1)]],
            tok_buf.at[pl.ds(_G0, _G1)], sem_tok).wait()

        @pl.when(i >= 1)
        def _wait_wb():
            pltpu.make_async_copy(
                wb_buf, out_hbm.at[bbase + cc], sem_wb).wait()

        @plsc.parallel_loop(0, _FULL // _L, 1)
        def _combine(g):
            c_vec = code_buf[pl.ds(g * _L, _L)]
            combine_rows(bufs, c_vec, g * _L, 0)

        c_vec = code_buf[pl.ds(_LEN - _L, _L)]
        combine_rows(bufs, c_vec, _LEN - _L, _L - (_LEN - _FULL))

        pltpu.async_copy(wb_buf, out_hbm.at[bbase + cc], sem_wb)

    issue_idx(0, sets[0])
    issue_idx(1, sets[1])
    issue_gather(sets[0])

    def pair(i, carry):
        for s in range(2):
            bufs = sets[s]
            cc = 2 * i + s

            @pl.when(cc + 1 < _BPW)
            def _gather_next():
                issue_gather(sets[1 - s])

            process(i, cc, bufs)

            @pl.when(cc + 2 < _BPW)
            def _idx_next():
                issue_idx(cc + 2, bufs)

        return carry

    lax.fori_loop(0, _HALF, pair, 0)
    for s in range(2):
        wb_buf, sem_wb = sets[s][1], sets[s][6]
        pltpu.make_async_copy(wb_buf, out_hbm.at[bbase], sem_wb).wait()


_sc_call = pl.kernel(
    _body,
    out_type=jax.ShapeDtypeStruct((_B, _LEN, _D), jnp.float32),
    mesh=plsc.VectorSubcoreMesh(core_axis_name="c", subcore_axis_name="s"),
    scratch_types=[
        pltpu.VMEM((_NSEG * _LEN * _D,), jnp.float32),
        pltpu.VMEM((_LEN, 2 * _D), jnp.float32),
        pltpu.VMEM((_LEN, _D), jnp.float32),
        pltpu.VMEM((_LEN,), jnp.int32),
        pltpu.VMEM((_LEN,), jnp.int32),
        pltpu.VMEM((_LEN, 2 * _D), jnp.float32),
        pltpu.VMEM((_LEN, _D), jnp.float32),
        pltpu.VMEM((_LEN,), jnp.int32),
        pltpu.VMEM((_LEN,), jnp.int32),
        pltpu.SemaphoreType.DMA,
        pltpu.SemaphoreType.DMA,
        pltpu.SemaphoreType.DMA,
        pltpu.SemaphoreType.DMA,
        pltpu.SemaphoreType.DMA,
        pltpu.SemaphoreType.DMA,
    ],
    compiler_params=pltpu.CompilerParams(use_tc_tiling_on_sc=True),
)


@jax.jit
def kernel(seq, seg, tok_table, seg_table):
    enc = _sinusoidal_encoding()
    addend = (seg_table[:, None, :] + enc[None, :, :]).reshape(
        _NSEG * _LEN * _D)
    seq_i = seq.astype(jnp.int32)
    seqp = (seq_i >> 1).reshape(_N)
    code = (seg.astype(jnp.int32) * _LEN
            + jnp.arange(_LEN, dtype=jnp.int32)[None, :]
            + (seq_i & 1) * 512).reshape(_N)
    tok2 = (tok_table + tok_table).reshape(_TOKP, 2 * _D)
    return _sc_call(seqp, code, tok2, addend)
